# Initial kernel scaffold; baseline (speedup 1.0000x reference)
import jax
import jax.numpy as jnp
from jax.experimental import pallas as pl


def kernel(x, W0_1, W1_1, b1, g1, be1, W0_2, W1_2, b2, W0_3, W1_3, b3, g3, be3, W0_4, W1_4, b4, edge_index, dataset_num):
    raise NotImplementedError("write your pallas kernel here")


# speedup vs baseline: 14.6190x; 14.6190x over previous
import functools

import jax
import jax.numpy as jnp
from jax import lax
from jax.experimental import pallas as pl
from jax.experimental.pallas import tpu as pltpu
from jax.experimental.pallas import tpu_sc as plsc

N = 10000
E = 320000
NPAD = 10112
NCORES = 2
NSUB = 16
NW = NCORES * NSUB
CH = 128
K = (E + NW * CH - 1) // (NW * CH)
EPAD = NW * K * CH
ROWS_PER_TILE = NPAD // NSUB

_mesh = plsc.VectorSubcoreMesh(core_axis_name="c", subcore_axis_name="s")


def _make_sc_scatter(w):
    @functools.partial(
        pl.kernel,
        mesh=_mesh,
        compiler_params=pltpu.CompilerParams(use_tc_tiling_on_sc=False),
        out_type=jax.ShapeDtypeStruct((NCORES, NPAD, w), jnp.float32),
        scratch_types=[
            pltpu.VMEM_SHARED((NPAD, w), jnp.float32),
            pltpu.VMEM((K, CH), jnp.int32),
            pltpu.VMEM((K, CH), jnp.int32),
            pltpu.VMEM((CH, w), jnp.float32),
            pltpu.SemaphoreType.DMA,
        ],
    )
    def sc_scatter(t_hbm, src_hbm, dst_hbm, z_hbm, out_hbm,
                   acc_sh, src_v, dst_v, rows_v, sem):
        cid = lax.axis_index("c")
        sid = lax.axis_index("s")
        wid = cid * NSUB + sid
        rs = pl.ds(sid * ROWS_PER_TILE, ROWS_PER_TILE)
        pltpu.sync_copy(z_hbm.at[rs], acc_sh.at[rs])
        pltpu.sync_copy(src_hbm.at[wid], src_v)
        pltpu.sync_copy(dst_hbm.at[wid], dst_v)
        plsc.subcore_barrier()

        def body(j, carry):
            pltpu.async_copy(t_hbm.at[src_v.at[j]], rows_v, sem).wait()
            pltpu.sync_copy(rows_v, acc_sh.at[dst_v.at[j]], add=True)
            return carry

        lax.fori_loop(0, K, body, 0)
        plsc.subcore_barrier()
        pltpu.sync_copy(acc_sh.at[rs], out_hbm.at[cid, rs])

    return sc_scatter


_sc_scatter_64 = _make_sc_scatter(64)
_sc_scatter_16 = _make_sc_scatter(16)


@functools.partial(
    pl.kernel,
    mesh=_mesh,
    compiler_params=pltpu.CompilerParams(use_tc_tiling_on_sc=False),
    out_type=jax.ShapeDtypeStruct((NCORES, NPAD, 16), jnp.float32),
    scratch_types=[
        pltpu.VMEM_SHARED((NPAD, 16), jnp.float32),
        pltpu.VMEM((K, CH), jnp.int32),
        pltpu.VMEM((CH, 16), jnp.float32),
    ],
)
def _sc_degree(src_hbm, z_hbm, ones_hbm, out_hbm, acc_sh, src_v, ones_v):
    cid = lax.axis_index("c")
    sid = lax.axis_index("s")
    wid = cid * NSUB + sid
    rs = pl.ds(sid * ROWS_PER_TILE, ROWS_PER_TILE)
    pltpu.sync_copy(z_hbm.at[rs], acc_sh.at[rs])
    pltpu.sync_copy(src_hbm.at[wid], src_v)
    pltpu.sync_copy(ones_hbm, ones_v)
    plsc.subcore_barrier()

    def body(j, carry):
        pltpu.sync_copy(ones_v, acc_sh.at[src_v.at[j]], add=True)
        return carry

    lax.fori_loop(0, K, body, 0)
    plsc.subcore_barrier()
    pltpu.sync_copy(acc_sh.at[rs], out_hbm.at[cid, rs])


def _pad_rows(t, nrows):
    return jnp.concatenate(
        [t, jnp.zeros((nrows - t.shape[0], t.shape[1]), t.dtype)], axis=0)


def _tc0_body(degp_ref, x_ref, w11_ref, dis_ref, t1_ref):
    deg = degp_ref[0, :, 0:1] + degp_ref[1, :, 0:1]
    safe = jnp.where(deg > 0, deg, 1.0)
    dis = jnp.where(deg > 0, lax.rsqrt(safe), 0.0)
    dis_ref[...] = jnp.broadcast_to(dis, (NPAD, 128))
    xw = jnp.dot(x_ref[...], w11_ref[...].T, preferred_element_type=jnp.float32)
    t1_ref[...] = _pad_rows(dis[:N] * xw, NPAD)


def _bn_relu(pre, g, be):
    m = jnp.mean(pre, axis=0)
    v = jnp.mean((pre - m) ** 2, axis=0)
    return jnp.maximum(g * (pre - m) / jnp.sqrt(v + 1e-5) + be, 0.0)


def _tc1_body(accp_ref, x_ref, w01_ref, b1_ref, g1_ref, be1_ref, dis_ref,
              w12_ref, x1_ref, t2_ref):
    acc = accp_ref[0, :N] + accp_ref[1, :N]
    pre = (jnp.dot(x_ref[...], w01_ref[...].T, preferred_element_type=jnp.float32)
           - dis_ref[:N, 0:64] * acc + b1_ref[...])
    x1 = _bn_relu(pre, g1_ref[...], be1_ref[...])
    x1_ref[...] = x1
    xw = jnp.dot(x1, w12_ref[...].T, preferred_element_type=jnp.float32)
    t2_ref[...] = _pad_rows(dis_ref[:N, 0:16] * xw, NPAD)


def _tc2_body(accp_ref, x1_ref, w02_ref, b2_ref, dis_ref, x2_ref, t3_ref):
    acc = accp_ref[0, :N] + accp_ref[1, :N]
    x2 = (jnp.dot(x1_ref[...], w02_ref[...].T, preferred_element_type=jnp.float32)
          - dis_ref[:N, 0:16] * acc + b2_ref[...])
    x2_ref[...] = x2
    t3_ref[...] = _pad_rows(dis_ref[:N, 0:16] * x2, NPAD)


def _tc3_body(accp_ref, x2_ref, w03_ref, w13_ref, b3_ref, g3_ref, be3_ref,
              dis_ref, x3_ref, t4_ref):
    acc = accp_ref[0, :N] + accp_ref[1, :N]
    tx1w = jnp.dot(-dis_ref[:N, 0:16] * acc, w13_ref[...].T,
                   preferred_element_type=jnp.float32)
    pre = (jnp.dot(x2_ref[...], w03_ref[...].T, preferred_element_type=jnp.float32)
           + tx1w + b3_ref[...])
    x3 = _bn_relu(pre, g3_ref[...], be3_ref[...])
    x3_ref[...] = x3
    t4_ref[...] = _pad_rows(dis_ref[:N, 0:64] * x3, NPAD)


def _tc4_body(accp_ref, x3_ref, w04_ref, w14_ref, b4_ref, dis_ref, out_ref):
    acc = accp_ref[0, :N] + accp_ref[1, :N]
    tx1w = jnp.dot(-dis_ref[:N, 0:64] * acc, w14_ref[...].T,
                   preferred_element_type=jnp.float32)
    out_ref[...] = (jnp.dot(x3_ref[...], w04_ref[...].T,
                            preferred_element_type=jnp.float32)
                    + tx1w + b4_ref[...])


def _sds(shape):
    return jax.ShapeDtypeStruct(shape, jnp.float32)


def kernel(x, W0_1, W1_1, b1, g1, be1, W0_2, W1_2, b2, W0_3, W1_3, b3,
           g3, be3, W0_4, W1_4, b4, edge_index, dataset_num):
    src = jnp.concatenate(
        [edge_index[0], jnp.full((EPAD - E,), N, jnp.int32)]).reshape(NW, K, CH)
    dst = jnp.concatenate(
        [edge_index[1], jnp.full((EPAD - E,), N, jnp.int32)]).reshape(NW, K, CH)
    z16 = jnp.zeros((NPAD, 16), jnp.float32)
    z64 = jnp.zeros((NPAD, 64), jnp.float32)
    ones = jnp.ones((CH, 16), jnp.float32)

    degp = _sc_degree(src, z16, ones)

    dis, t1 = pl.pallas_call(
        _tc0_body, out_shape=[_sds((NPAD, 128)), _sds((NPAD, 64))],
    )(degp, x, W1_1)

    acc1 = _sc_scatter_64(t1, src, dst, z64)
    x1, t2 = pl.pallas_call(
        _tc1_body, out_shape=[_sds((N, 64)), _sds((NPAD, 16))],
    )(acc1, x, W0_1, b1, g1, be1, dis, W1_2)

    acc2 = _sc_scatter_16(t2, src, dst, z16)
    x2, t3 = pl.pallas_call(
        _tc2_body, out_shape=[_sds((N, 16)), _sds((NPAD, 16))],
    )(acc2, x1, W0_2, b2, dis)

    acc3 = _sc_scatter_16(t3, src, dst, z16)
    x3, t4 = pl.pallas_call(
        _tc3_body, out_shape=[_sds((N, 64)), _sds((NPAD, 64))],
    )(acc3, x2, W0_3, W1_3, b3, g3, be3, dis)

    acc4 = _sc_scatter_64(t4, src, dst, z64)
    x4 = pl.pallas_call(
        _tc4_body, out_shape=_sds((N, 128)),
    )(acc4, x3, W0_4, W1_4, b4, dis)
    return x4


# speedup vs baseline: 18.9239x; 1.2945x over previous
import functools

import jax
import jax.numpy as jnp
from jax import lax
from jax.experimental import pallas as pl
from jax.experimental.pallas import tpu as pltpu
from jax.experimental.pallas import tpu_sc as plsc

N = 10000
E = 320000
NPAD = 10112
NCORES = 2
NSUB = 16
NW = NCORES * NSUB
CH = 128
K = (E + NW * CH - 1) // (NW * CH)
EPAD = NW * K * CH
ROWS_PER_TILE = NPAD // NSUB

_mesh = plsc.VectorSubcoreMesh(core_axis_name="c", subcore_axis_name="s")


def _make_sc_scatter(w):
    @functools.partial(
        pl.kernel,
        mesh=_mesh,
        compiler_params=pltpu.CompilerParams(use_tc_tiling_on_sc=False),
        out_type=jax.ShapeDtypeStruct((NCORES, NPAD, w), jnp.float32),
        scratch_types=[
            pltpu.VMEM_SHARED((NPAD, w), jnp.float32),
            pltpu.VMEM((K, CH), jnp.int32),
            pltpu.VMEM((K, CH), jnp.int32),
            pltpu.VMEM((CH, w), jnp.float32),
            pltpu.VMEM((CH, w), jnp.float32),
            pltpu.SemaphoreType.DMA,
            pltpu.SemaphoreType.DMA,
        ],
    )
    def sc_scatter(t_hbm, src_hbm, dst_hbm, z_hbm, out_hbm,
                   acc_sh, src_v, dst_v, rows0_v, rows1_v, sem0, sem1):
        cid = lax.axis_index("c")
        sid = lax.axis_index("s")
        wid = cid * NSUB + sid
        rs = pl.ds(sid * ROWS_PER_TILE, ROWS_PER_TILE)
        pltpu.sync_copy(z_hbm.at[rs], acc_sh.at[rs])
        pltpu.sync_copy(src_hbm.at[wid], src_v)
        pltpu.sync_copy(dst_hbm.at[wid], dst_v)
        plsc.subcore_barrier()
        bufs = ((rows0_v, sem0), (rows1_v, sem1))
        pltpu.async_copy(t_hbm.at[src_v.at[0]], rows0_v, sem0)

        def body(j, carry):
            for par in range(2):
                rows_v, sem = bufs[par]
                nrows_v, nsem = bufs[1 - par]

                @pl.when(lax.rem(j, 2) == par)
                def _():
                    @pl.when(j + 1 < K)
                    def _():
                        pltpu.async_copy(t_hbm.at[src_v.at[j + 1]], nrows_v,
                                         nsem)
                    pltpu.make_async_copy(t_hbm.at[src_v.at[j]], rows_v,
                                          sem).wait()
                    pltpu.sync_copy(rows_v, acc_sh.at[dst_v.at[j]], add=True)

            return carry

        lax.fori_loop(0, K, body, 0)
        plsc.subcore_barrier()
        pltpu.sync_copy(acc_sh.at[rs], out_hbm.at[cid, rs])

    return sc_scatter


_sc_scatter_64 = _make_sc_scatter(64)
_sc_scatter_16 = _make_sc_scatter(16)


@functools.partial(
    pl.kernel,
    mesh=_mesh,
    compiler_params=pltpu.CompilerParams(use_tc_tiling_on_sc=False),
    out_type=jax.ShapeDtypeStruct((NCORES, NPAD, 16), jnp.float32),
    scratch_types=[
        pltpu.VMEM_SHARED((NPAD, 16), jnp.float32),
        pltpu.VMEM((K, CH), jnp.int32),
        pltpu.VMEM((CH, 16), jnp.float32),
    ],
)
def _sc_degree(src_hbm, z_hbm, ones_hbm, out_hbm, acc_sh, src_v, ones_v):
    cid = lax.axis_index("c")
    sid = lax.axis_index("s")
    wid = cid * NSUB + sid
    rs = pl.ds(sid * ROWS_PER_TILE, ROWS_PER_TILE)
    pltpu.sync_copy(z_hbm.at[rs], acc_sh.at[rs])
    pltpu.sync_copy(src_hbm.at[wid], src_v)
    pltpu.sync_copy(ones_hbm, ones_v)
    plsc.subcore_barrier()

    def body(j, carry):
        pltpu.sync_copy(ones_v, acc_sh.at[src_v.at[j]], add=True)
        return carry

    lax.fori_loop(0, K, body, 0)
    plsc.subcore_barrier()
    pltpu.sync_copy(acc_sh.at[rs], out_hbm.at[cid, rs])


def _pad_rows(t, nrows):
    return jnp.concatenate(
        [t, jnp.zeros((nrows - t.shape[0], t.shape[1]), t.dtype)], axis=0)


def _tc0_body(degp_ref, x_ref, w11_ref, dis_ref, t1_ref):
    deg = degp_ref[0, :, 0:1] + degp_ref[1, :, 0:1]
    safe = jnp.where(deg > 0, deg, 1.0)
    dis = jnp.where(deg > 0, lax.rsqrt(safe), 0.0)
    dis_ref[...] = jnp.broadcast_to(dis, (NPAD, 128))
    xw = jnp.dot(x_ref[...], w11_ref[...].T, preferred_element_type=jnp.float32)
    t1_ref[...] = _pad_rows(dis[:N] * xw, NPAD)


def _bn_relu(pre, g, be):
    m = jnp.mean(pre, axis=0)
    v = jnp.mean((pre - m) ** 2, axis=0)
    return jnp.maximum(g * (pre - m) / jnp.sqrt(v + 1e-5) + be, 0.0)


def _tc1_body(accp_ref, x_ref, w01_ref, b1_ref, g1_ref, be1_ref, dis_ref,
              w12_ref, x1_ref, t2_ref):
    acc = accp_ref[0, :N] + accp_ref[1, :N]
    pre = (jnp.dot(x_ref[...], w01_ref[...].T, preferred_element_type=jnp.float32)
           - dis_ref[:N, 0:64] * acc + b1_ref[...])
    x1 = _bn_relu(pre, g1_ref[...], be1_ref[...])
    x1_ref[...] = x1
    xw = jnp.dot(x1, w12_ref[...].T, preferred_element_type=jnp.float32)
    t2_ref[...] = _pad_rows(dis_ref[:N, 0:16] * xw, NPAD)


def _tc2_body(accp_ref, x1_ref, w02_ref, b2_ref, dis_ref, x2_ref, t3_ref):
    acc = accp_ref[0, :N] + accp_ref[1, :N]
    x2 = (jnp.dot(x1_ref[...], w02_ref[...].T, preferred_element_type=jnp.float32)
          - dis_ref[:N, 0:16] * acc + b2_ref[...])
    x2_ref[...] = x2
    t3_ref[...] = _pad_rows(dis_ref[:N, 0:16] * x2, NPAD)


def _tc3_body(accp_ref, x2_ref, w03_ref, w13_ref, b3_ref, g3_ref, be3_ref,
              dis_ref, x3_ref, t4_ref):
    acc = accp_ref[0, :N] + accp_ref[1, :N]
    tx1w = jnp.dot(-dis_ref[:N, 0:16] * acc, w13_ref[...].T,
                   preferred_element_type=jnp.float32)
    pre = (jnp.dot(x2_ref[...], w03_ref[...].T, preferred_element_type=jnp.float32)
           + tx1w + b3_ref[...])
    x3 = _bn_relu(pre, g3_ref[...], be3_ref[...])
    x3_ref[...] = x3
    t4_ref[...] = _pad_rows(dis_ref[:N, 0:64] * x3, NPAD)


def _tc4_body(accp_ref, x3_ref, w04_ref, w14_ref, b4_ref, dis_ref, out_ref):
    acc = accp_ref[0, :N] + accp_ref[1, :N]
    tx1w = jnp.dot(-dis_ref[:N, 0:64] * acc, w14_ref[...].T,
                   preferred_element_type=jnp.float32)
    out_ref[...] = (jnp.dot(x3_ref[...], w04_ref[...].T,
                            preferred_element_type=jnp.float32)
                    + tx1w + b4_ref[...])


def _sds(shape):
    return jax.ShapeDtypeStruct(shape, jnp.float32)


def kernel(x, W0_1, W1_1, b1, g1, be1, W0_2, W1_2, b2, W0_3, W1_3, b3,
           g3, be3, W0_4, W1_4, b4, edge_index, dataset_num):
    src = jnp.concatenate(
        [edge_index[0], jnp.full((EPAD - E,), N, jnp.int32)]).reshape(NW, K, CH)
    dst = jnp.concatenate(
        [edge_index[1], jnp.full((EPAD - E,), N, jnp.int32)]).reshape(NW, K, CH)
    z16 = jnp.zeros((NPAD, 16), jnp.float32)
    z64 = jnp.zeros((NPAD, 64), jnp.float32)
    ones = jnp.ones((CH, 16), jnp.float32)

    degp = _sc_degree(src, z16, ones)

    dis, t1 = pl.pallas_call(
        _tc0_body, out_shape=[_sds((NPAD, 128)), _sds((NPAD, 64))],
    )(degp, x, W1_1)

    acc1 = _sc_scatter_64(t1, src, dst, z64)
    x1, t2 = pl.pallas_call(
        _tc1_body, out_shape=[_sds((N, 64)), _sds((NPAD, 16))],
    )(acc1, x, W0_1, b1, g1, be1, dis, W1_2)

    acc2 = _sc_scatter_16(t2, src, dst, z16)
    x2, t3 = pl.pallas_call(
        _tc2_body, out_shape=[_sds((N, 16)), _sds((NPAD, 16))],
    )(acc2, x1, W0_2, b2, dis)

    acc3 = _sc_scatter_16(t3, src, dst, z16)
    x3, t4 = pl.pallas_call(
        _tc3_body, out_shape=[_sds((N, 64)), _sds((NPAD, 64))],
    )(acc3, x2, W0_3, W1_3, b3, g3, be3, dis)

    acc4 = _sc_scatter_64(t4, src, dst, z64)
    x4 = pl.pallas_call(
        _tc4_body, out_shape=_sds((N, 128)),
    )(acc4, x3, W0_4, W1_4, b4, dis)
    return x4
